# baseline (speedup 1.0000x reference)
import functools

import jax
import jax.numpy as jnp
from jax import lax
from jax.experimental import pallas as pl
from jax.experimental.pallas import tpu as pltpu
from jax.experimental.pallas import tpu_sc as plsc

L = 16
NC = 2
NS = 16
NW = NC * NS
DIM = 64
NTYPES = 3
S = 50
B = 16384
BW = B // NW
CB = 256
CBP = CB + 8
NJ = BW // CB


def _sc_body(nums_hbm, types_hbm, ntab_hbm, ttab_hbm, out_hbm,
             idx_v, tidx_v, ttab_v, rows_a, rows_b, trans_a, trans_b,
             gsa, gsb, ssa, ssb):
    c = lax.axis_index("c")
    s_ax = lax.axis_index("s")
    wid = s_ax * NC + c
    b0 = wid * BW

    pltpu.sync_copy(nums_hbm.at[:, pl.ds(b0, BW)], idx_v)
    pltpu.sync_copy(types_hbm.at[:, pl.ds(b0, BW)], tidx_v)
    pltpu.sync_copy(ttab_hbm, ttab_v)

    def gather(si, j, rows, sem):
        return pltpu.async_copy(
            ntab_hbm.at[idx_v.at[si, pl.ds(j * CB, CB)]], rows, sem)

    def gather_wait(si, j, rows, sem):
        pltpu.make_async_copy(
            ntab_hbm.at[idx_v.at[si, pl.ds(j * CB, CB)]], rows, sem).wait()

    def store(si, j, trans, sem):
        return pltpu.async_copy(
            trans.at[:, pl.ds(0, CB)],
            out_hbm.at[si, :, pl.ds(b0 + j * CB, CB)], sem)

    def store_wait(si, j, trans, sem):
        pltpu.make_async_copy(
            trans.at[:, pl.ds(0, CB)],
            out_hbm.at[si, :, pl.ds(b0 + j * CB, CB)], sem).wait()

    cidx = [lax.iota(jnp.int32, L) + g * L for g in range(DIM // L)]
    trow = [[ttab_v[t, pl.ds(g * L, L)] for g in range(DIM // L)]
            for t in range(NTYPES)]

    def compute(si, j, rows, trans):
        @plsc.parallel_loop(0, CB // L, unroll=4)
        def grp_body(g16):
            bsl = pl.ds(j * CB + g16 * L, L)
            t16 = tidx_v[si, bsl]
            for jj in range(L):
                t = t16[jj]
                p0 = t == 0
                p1 = t == 1
                r = g16 * L + jj
                rsp = jnp.full((L,), r, jnp.int32)
                for g in range(DIM // L):
                    sl = pl.ds(g * L, L)
                    add = jnp.where(p0, trow[0][g],
                                    jnp.where(p1, trow[1][g], trow[2][g]))
                    v = rows[r, sl] + add
                    plsc.store_scatter(trans, [cidx[g], rsp], v)

    last = S - 1

    gather(0, 0, rows_a, gsa)
    gather(0, 1, rows_b, gsb)
    store(last, 0, trans_a, ssa)
    store(last, 1, trans_b, ssb)

    def s_body(si, carry):
        with jax.named_scope("gwait"):
            gather_wait(si, 0, rows_a, gsa)
        with jax.named_scope("swait"):
            store_wait(last, 0, trans_a, ssa)
        with jax.named_scope("compute"):
            compute(si, 0, rows_a, trans_a)
        gather(jnp.minimum(si + 1, last), 0, rows_a, gsa)
        store(si, 0, trans_a, ssa)
        with jax.named_scope("gwait"):
            gather_wait(si, 1, rows_b, gsb)
        with jax.named_scope("swait"):
            store_wait(last, 1, trans_b, ssb)
        with jax.named_scope("compute"):
            compute(si, 1, rows_b, trans_b)
        gather(jnp.minimum(si + 1, last), 1, rows_b, gsb)
        store(si, 1, trans_b, ssb)
        return carry

    lax.fori_loop(0, S, s_body, 0)

    store_wait(last, 0, trans_a, ssa)
    store_wait(last, 1, trans_b, ssb)
    gather_wait(last, 0, rows_a, gsa)
    gather_wait(last, 1, rows_b, gsb)


def kernel(task_nums, task_types, task_num_table, task_type_table):
    nums_t = task_nums.astype(jnp.int32).T
    types_t = task_types.astype(jnp.int32).T

    mesh = plsc.VectorSubcoreMesh(core_axis_name="c", subcore_axis_name="s")
    call = pl.kernel(
        _sc_body,
        out_type=jax.ShapeDtypeStruct((S, DIM, B), jnp.float32),
        mesh=mesh,
        scratch_types=[
            pltpu.VMEM((S, BW), jnp.int32),
            pltpu.VMEM((S, BW), jnp.int32),
            pltpu.VMEM((NTYPES, DIM), jnp.float32),
            pltpu.VMEM((CB, DIM), jnp.float32),
            pltpu.VMEM((CB, DIM), jnp.float32),
            pltpu.VMEM((DIM, CBP), jnp.float32),
            pltpu.VMEM((DIM, CBP), jnp.float32),
            pltpu.SemaphoreType.DMA,
            pltpu.SemaphoreType.DMA,
            pltpu.SemaphoreType.DMA,
            pltpu.SemaphoreType.DMA,
        ],
        compiler_params=pltpu.CompilerParams(
            use_tc_tiling_on_sc=False, needs_layout_passes=False),
    )
    out = call(nums_t, types_t, task_num_table, task_type_table)
    return out.transpose(2, 0, 1)


# speedup vs baseline: 1.0807x; 1.0807x over previous
import functools

import jax
import jax.numpy as jnp
from jax import lax
from jax.experimental import pallas as pl
from jax.experimental.pallas import tpu as pltpu
from jax.experimental.pallas import tpu_sc as plsc

L = 16
NC = 2
NS = 16
NW = NC * NS
DIM = 64
NTYPES = 3
S = 50
B = 16384
BW = B // NW
CB = 256
CBP = CB + 8
NJ = BW // CB


def _sc_body(nums_hbm, types_hbm, ntab_hbm, ttab_hbm, out_hbm,
             idx_v, tidx_v, ttab_v, rows_a, rows_b, trans_a, trans_b,
             gsa, gsb, ssa, ssb):
    c = lax.axis_index("c")
    s_ax = lax.axis_index("s")
    wid = s_ax * NC + c
    b0 = wid * BW

    pltpu.sync_copy(nums_hbm.at[:, pl.ds(b0, BW)], idx_v)
    pltpu.sync_copy(types_hbm.at[:, pl.ds(b0, BW)], tidx_v)
    pltpu.sync_copy(ttab_hbm, ttab_v)

    def gather(si, j, rows, sem):
        return pltpu.async_copy(
            ntab_hbm.at[idx_v.at[si, pl.ds(j * CB, CB)]], rows, sem)

    def gather_wait(si, j, rows, sem):
        pltpu.make_async_copy(
            ntab_hbm.at[idx_v.at[si, pl.ds(j * CB, CB)]], rows, sem).wait()

    def store(si, j, trans, sem):
        return pltpu.async_copy(
            trans.at[:, pl.ds(0, CB)],
            out_hbm.at[si, :, pl.ds(b0 + j * CB, CB)], sem)

    def store_wait(si, j, trans, sem):
        pltpu.make_async_copy(
            trans.at[:, pl.ds(0, CB)],
            out_hbm.at[si, :, pl.ds(b0 + j * CB, CB)], sem).wait()

    cidx = [lax.iota(jnp.int32, L) + g * L for g in range(DIM // L)]
    trow = [[ttab_v[t, pl.ds(g * L, L)] for g in range(DIM // L)]
            for t in range(NTYPES)]

    def compute(si, j, rows, trans):
        @plsc.parallel_loop(0, CB // L, unroll=2)
        def grp_body(g16):
            bsl = pl.ds(j * CB + g16 * L, L)
            t16 = tidx_v[si, bsl]
            for jj in range(L):
                t = t16[jj]
                p0 = t == 0
                p1 = t == 1
                r = g16 * L + jj
                rsp = jnp.full((L,), r, jnp.int32)
                for g in range(DIM // L):
                    sl = pl.ds(g * L, L)
                    add = jnp.where(p0, trow[0][g],
                                    jnp.where(p1, trow[1][g], trow[2][g]))
                    v = rows[r, sl] + add
                    plsc.store_scatter(trans, [cidx[g], rsp], v)

    last = S - 1

    gather(0, 0, rows_a, gsa)
    gather(0, 1, rows_b, gsb)
    store(last, 0, trans_a, ssa)
    store(last, 1, trans_b, ssb)

    def s_body(si, carry):
        with jax.named_scope("gwait"):
            gather_wait(si, 0, rows_a, gsa)
        with jax.named_scope("swait"):
            store_wait(last, 0, trans_a, ssa)
        with jax.named_scope("compute"):
            compute(si, 0, rows_a, trans_a)
        gather(jnp.minimum(si + 1, last), 0, rows_a, gsa)
        store(si, 0, trans_a, ssa)
        with jax.named_scope("gwait"):
            gather_wait(si, 1, rows_b, gsb)
        with jax.named_scope("swait"):
            store_wait(last, 1, trans_b, ssb)
        with jax.named_scope("compute"):
            compute(si, 1, rows_b, trans_b)
        gather(jnp.minimum(si + 1, last), 1, rows_b, gsb)
        store(si, 1, trans_b, ssb)
        return carry

    lax.fori_loop(0, S, s_body, 0)

    store_wait(last, 0, trans_a, ssa)
    store_wait(last, 1, trans_b, ssb)
    gather_wait(last, 0, rows_a, gsa)
    gather_wait(last, 1, rows_b, gsb)


def kernel(task_nums, task_types, task_num_table, task_type_table):
    nums_t = task_nums.astype(jnp.int32).T
    types_t = task_types.astype(jnp.int32).T

    mesh = plsc.VectorSubcoreMesh(core_axis_name="c", subcore_axis_name="s")
    call = pl.kernel(
        _sc_body,
        out_type=jax.ShapeDtypeStruct((S, DIM, B), jnp.float32),
        mesh=mesh,
        scratch_types=[
            pltpu.VMEM((S, BW), jnp.int32),
            pltpu.VMEM((S, BW), jnp.int32),
            pltpu.VMEM((NTYPES, DIM), jnp.float32),
            pltpu.VMEM((CB, DIM), jnp.float32),
            pltpu.VMEM((CB, DIM), jnp.float32),
            pltpu.VMEM((DIM, CBP), jnp.float32),
            pltpu.VMEM((DIM, CBP), jnp.float32),
            pltpu.SemaphoreType.DMA,
            pltpu.SemaphoreType.DMA,
            pltpu.SemaphoreType.DMA,
            pltpu.SemaphoreType.DMA,
        ],
        compiler_params=pltpu.CompilerParams(
            use_tc_tiling_on_sc=False, needs_layout_passes=False),
    )
    out = call(nums_t, types_t, task_num_table, task_type_table)
    return out.transpose(2, 0, 1)


# speedup vs baseline: 1.0921x; 1.0105x over previous
import functools

import jax
import jax.numpy as jnp
from jax import lax
from jax.experimental import pallas as pl
from jax.experimental.pallas import tpu as pltpu
from jax.experimental.pallas import tpu_sc as plsc

L = 16
NC = 2
NS = 16
NW = NC * NS
DIM = 64
NTYPES = 3
S = 50
B = 16384
BW = B // NW
CB = 256
CBP = CB + 8
NJ = BW // CB


def _sc_body(nums_hbm, types_hbm, ntab_hbm, ttab_hbm, out_hbm,
             idx_v, tidx_v, ttab_v, rows_a, rows_b, trans_a, trans_b,
             gsa, gsb, ssa, ssb):
    c = lax.axis_index("c")
    s_ax = lax.axis_index("s")
    wid = s_ax * NC + c
    b0 = wid * BW

    pltpu.sync_copy(nums_hbm.at[:, pl.ds(b0, BW)], idx_v)
    pltpu.sync_copy(types_hbm.at[:, pl.ds(b0, BW)], tidx_v)
    pltpu.sync_copy(ttab_hbm, ttab_v)

    def gather(si, j, rows, sem):
        return pltpu.async_copy(
            ntab_hbm.at[idx_v.at[si, pl.ds(j * CB, CB)]], rows, sem)

    def gather_wait(si, j, rows, sem):
        pltpu.make_async_copy(
            ntab_hbm.at[idx_v.at[si, pl.ds(j * CB, CB)]], rows, sem).wait()

    def store(si, j, trans, sem):
        return pltpu.async_copy(
            trans.at[:, pl.ds(0, CB)],
            out_hbm.at[si, :, pl.ds(b0 + j * CB, CB)], sem)

    def store_wait(si, j, trans, sem):
        pltpu.make_async_copy(
            trans.at[:, pl.ds(0, CB)],
            out_hbm.at[si, :, pl.ds(b0 + j * CB, CB)], sem).wait()

    cidx = [lax.iota(jnp.int32, L) + g * L for g in range(DIM // L)]
    trow = [[ttab_v[t, pl.ds(g * L, L)] for g in range(DIM // L)]
            for t in range(NTYPES)]

    def compute(si, j, rows, trans):
        @plsc.parallel_loop(0, CB // L, unroll=2)
        def grp_body(g16):
            bsl = pl.ds(j * CB + g16 * L, L)
            t16 = tidx_v[si, bsl]
            for jj in range(L):
                t = t16[jj]
                p0 = t == 0
                p1 = t == 1
                r = g16 * L + jj
                rsp = jnp.full((L,), r, jnp.int32)
                for g in range(DIM // L):
                    sl = pl.ds(g * L, L)
                    add = jnp.where(p0, trow[0][g],
                                    jnp.where(p1, trow[1][g], trow[2][g]))
                    v = rows[r, sl] + add
                    plsc.store_scatter(trans, [cidx[g], rsp], v)

    last = S - 1

    gather(0, 0, rows_a, gsa)
    gather(0, 1, rows_b, gsb)
    store(last, 0, trans_a, ssa)
    store(last, 1, trans_b, ssb)

    def s_body(si, carry):
        gather_wait(si, 0, rows_a, gsa)
        store_wait(last, 0, trans_a, ssa)
        compute(si, 0, rows_a, trans_a)
        gather(jnp.minimum(si + 1, last), 0, rows_a, gsa)
        store(si, 0, trans_a, ssa)
        gather_wait(si, 1, rows_b, gsb)
        store_wait(last, 1, trans_b, ssb)
        compute(si, 1, rows_b, trans_b)
        gather(jnp.minimum(si + 1, last), 1, rows_b, gsb)
        store(si, 1, trans_b, ssb)
        return carry

    lax.fori_loop(0, S, s_body, 0)

    store_wait(last, 0, trans_a, ssa)
    store_wait(last, 1, trans_b, ssb)
    gather_wait(last, 0, rows_a, gsa)
    gather_wait(last, 1, rows_b, gsb)


def kernel(task_nums, task_types, task_num_table, task_type_table):
    nums_t = task_nums.astype(jnp.int32).T
    types_t = task_types.astype(jnp.int32).T

    mesh = plsc.VectorSubcoreMesh(core_axis_name="c", subcore_axis_name="s")
    call = pl.kernel(
        _sc_body,
        out_type=jax.ShapeDtypeStruct((S, DIM, B), jnp.float32),
        mesh=mesh,
        scratch_types=[
            pltpu.VMEM((S, BW), jnp.int32),
            pltpu.VMEM((S, BW), jnp.int32),
            pltpu.VMEM((NTYPES, DIM), jnp.float32),
            pltpu.VMEM((CB, DIM), jnp.float32),
            pltpu.VMEM((CB, DIM), jnp.float32),
            pltpu.VMEM((DIM, CBP), jnp.float32),
            pltpu.VMEM((DIM, CBP), jnp.float32),
            pltpu.SemaphoreType.DMA,
            pltpu.SemaphoreType.DMA,
            pltpu.SemaphoreType.DMA,
            pltpu.SemaphoreType.DMA,
        ],
        compiler_params=pltpu.CompilerParams(
            use_tc_tiling_on_sc=False, needs_layout_passes=False),
    )
    out = call(nums_t, types_t, task_num_table, task_type_table)
    return out.transpose(2, 0, 1)
